# baseline (speedup 1.0000x reference)
import functools

import jax
import jax.numpy as jnp
from jax import lax
from jax.experimental import pallas as pl
from jax.experimental.pallas import tpu as pltpu
from jax.experimental.pallas import tpu_sc as plsc

VOCAB_ROWS = 100002
DIM = 64
B = 16384
L = 50
N = B * L
N2 = N // 2

NUM_CORES = 2
NUM_SUBCORES = 16
NW = NUM_CORES * NUM_SUBCORES
PER_W = N2 // NW
CHUNK = 320
NCHUNK = PER_W // CHUNK
NPAIR = NCHUNK // 2


def _make_kernel():
  mesh = plsc.VectorSubcoreMesh(core_axis_name="c", subcore_axis_name="s")

  @functools.partial(
      pl.kernel,
      mesh=mesh,
      compiler_params=pltpu.CompilerParams(use_tc_tiling_on_sc=False),
      out_type=jax.ShapeDtypeStruct((N2, 2 * DIM), jnp.float32),
      scratch_types=[
          pltpu.VMEM((PER_W,), jnp.int32),
          pltpu.VMEM((PER_W,), jnp.int32),
          pltpu.VMEM((CHUNK, DIM), jnp.float32),
          pltpu.VMEM((CHUNK, DIM), jnp.float32),
          pltpu.VMEM((CHUNK, DIM), jnp.float32),
          pltpu.VMEM((CHUNK, DIM), jnp.float32),
          pltpu.SemaphoreType.DMA,
          pltpu.SemaphoreType.DMA,
          pltpu.SemaphoreType.DMA,
          pltpu.SemaphoreType.DMA,
      ],
  )
  def gather_kernel(idx_e_hbm, idx_o_hbm, table_hbm, out2_hbm,
                    idx_e_v, idx_o_v, rows0e, rows0o, rows1e, rows1o,
                    gsem0, gsem1, wsem0, wsem1):
    wid = lax.axis_index("s") * NUM_CORES + lax.axis_index("c")
    base = wid * PER_W
    pltpu.sync_copy(idx_e_hbm.at[pl.ds(base, PER_W)], idx_e_v)
    pltpu.sync_copy(idx_o_hbm.at[pl.ds(base, PER_W)], idx_o_v)

    def start_gather(g, bufe, bufo, sem):
      sl = pl.ds(g * CHUNK, CHUNK)
      pltpu.async_copy(table_hbm.at[idx_e_v.at[sl]], bufe, sem)
      pltpu.async_copy(table_hbm.at[idx_o_v.at[sl]], bufo, sem)

    def wait_gather(g, bufe, bufo, sem):
      sl = pl.ds(g * CHUNK, CHUNK)
      pltpu.make_async_copy(table_hbm.at[idx_e_v.at[sl]], bufe, sem).wait()
      pltpu.make_async_copy(table_hbm.at[idx_o_v.at[sl]], bufo, sem).wait()

    def start_write(g, bufe, bufo, sem):
      sl = pl.ds(base + g * CHUNK, CHUNK)
      pltpu.async_copy(bufe, out2_hbm.at[sl, pl.ds(0, DIM)], sem)
      pltpu.async_copy(bufo, out2_hbm.at[sl, pl.ds(DIM, DIM)], sem)

    def wait_write(g, bufe, bufo, sem):
      sl = pl.ds(base + g * CHUNK, CHUNK)
      pltpu.make_async_copy(bufe, out2_hbm.at[sl, pl.ds(0, DIM)], sem).wait()
      pltpu.make_async_copy(bufo, out2_hbm.at[sl, pl.ds(DIM, DIM)], sem).wait()

    start_gather(0, rows0e, rows0o, gsem0)
    start_gather(1, rows1e, rows1o, gsem1)

    def body(i, carry):
      g0 = 2 * i
      g1 = g0 + 1
      wait_gather(g0, rows0e, rows0o, gsem0)
      start_write(g0, rows0e, rows0o, wsem0)
      wait_gather(g1, rows1e, rows1o, gsem1)
      start_write(g1, rows1e, rows1o, wsem1)
      wait_write(g0, rows0e, rows0o, wsem0)
      start_gather(g0 + 2, rows0e, rows0o, gsem0)
      wait_write(g1, rows1e, rows1o, wsem1)
      start_gather(g1 + 2, rows1e, rows1o, gsem1)
      return carry

    lax.fori_loop(0, NPAIR - 1, body, 0)

    gl0 = NCHUNK - 2
    gl1 = NCHUNK - 1
    wait_gather(gl0, rows0e, rows0o, gsem0)
    start_write(gl0, rows0e, rows0o, wsem0)
    wait_gather(gl1, rows1e, rows1o, gsem1)
    start_write(gl1, rows1e, rows1o, wsem1)
    wait_write(gl0, rows0e, rows0o, wsem0)
    wait_write(gl1, rows1e, rows1o, wsem1)

  return gather_kernel


_gather = _make_kernel()


@jax.jit
def kernel(idx_input, table):
  idx_flat = idx_input.reshape(-1).astype(jnp.int32)
  idx_e = idx_flat[0::2]
  idx_o = idx_flat[1::2]
  out2 = _gather(idx_e, idx_o, table)
  return out2.reshape(B, L, DIM)
